# baseline (speedup 1.0000x reference)
import functools

import jax
import jax.numpy as jnp
from jax import lax
from jax.experimental import pallas as pl
from jax.experimental.pallas import tpu as pltpu
from jax.experimental.pallas import tpu_sc as plsc

K = 16
NEG = -3.0e38
IBIG = 2147483647

SC_NC = 2
SC_NS = 16
SC_NW = SC_NC * SC_NS


def _row_block(n):
    for rb in (400, 200, 80, 40, 16, 8):
        if n % rb == 0:
            return rb
    return n


def _topk_body(xn_ref, xnall_ref, x_ref, w_ref, vals_ref, idx_ref,
               dinv_ref, hs_ref):
    xn = xn_ref[...]
    sim = lax.dot_general(
        xn, xnall_ref[...],
        dimension_numbers=(((1,), (1,)), ((), ())),
        preferred_element_type=jnp.float32,
    )
    b, n = sim.shape
    colid = lax.broadcasted_iota(jnp.int32, (b, n), 1)
    vcols = []
    icols = []
    for _ in range(K):
        m = jnp.max(sim, axis=1, keepdims=True)
        cand = jnp.where(sim == m, colid, IBIG)
        ix = jnp.min(cand, axis=1, keepdims=True)
        sim = jnp.where(colid == ix, NEG, sim)
        vcols.append(m)
        icols.append(ix)
    vals = jnp.concatenate(vcols, axis=1)
    idx = jnp.concatenate(icols, axis=1)
    vals_ref[...] = vals
    idx_ref[...] = idx
    deg = 1.0 + jnp.sum(vals, axis=1, keepdims=True)
    dinv = jnp.where(deg > 0, lax.rsqrt(deg), 0.0)
    dinv_ref[...] = dinv
    h = jnp.dot(x_ref[...], w_ref[...], preferred_element_type=jnp.float32)
    hs_ref[...] = dinv * h


def _topk(xn, x, w):
    n, seq = xn.shape
    hid = w.shape[1]
    rb = 80 if n % 80 == 0 else _row_block(n)
    grid = n // rb
    return pl.pallas_call(
        _topk_body,
        grid=(grid,),
        in_specs=[
            pl.BlockSpec((rb, seq), lambda i: (i, 0)),
            pl.BlockSpec((n, seq), lambda i: (0, 0)),
            pl.BlockSpec((rb, seq), lambda i: (i, 0)),
            pl.BlockSpec((seq, hid), lambda i: (0, 0)),
        ],
        out_specs=[
            pl.BlockSpec((rb, K), lambda i: (i, 0)),
            pl.BlockSpec((rb, K), lambda i: (i, 0)),
            pl.BlockSpec((rb, 1), lambda i: (i, 0)),
            pl.BlockSpec((rb, hid), lambda i: (i, 0)),
        ],
        out_shape=[
            jax.ShapeDtypeStruct((n, K), jnp.float32),
            jax.ShapeDtypeStruct((n, K), jnp.int32),
            jax.ShapeDtypeStruct((n, 1), jnp.float32),
            jax.ShapeDtypeStruct((n, hid), jnp.float32),
        ],
    )(xn, xn, x, w)


def _gcn_gather(hs, idxf, valsf, np_rows):
    d = hs.shape[1]
    per_w = np_rows // SC_NW
    G = 4
    ngroups = per_w // G
    npairs = ngroups // 2
    nchunk = d // 16
    mesh = plsc.VectorSubcoreMesh(core_axis_name="c", subcore_axis_name="s")

    @functools.partial(
        pl.kernel,
        mesh=mesh,
        out_type=jax.ShapeDtypeStruct((np_rows, d), jnp.float32),
        scratch_types=[
            pltpu.VMEM((per_w * K,), jnp.int32),
            pltpu.VMEM((per_w * K,), jnp.float32),
            pltpu.VMEM((G * K, d), jnp.float32),
            pltpu.VMEM((G * K, d), jnp.float32),
            pltpu.VMEM((G, d), jnp.float32),
            pltpu.VMEM((G, d), jnp.float32),
            pltpu.SemaphoreType.DMA,
            pltpu.SemaphoreType.DMA,
            pltpu.SemaphoreType.DMA,
            pltpu.SemaphoreType.DMA,
        ],
    )
    def sc_kernel(hs_hbm, idx_hbm, vals_hbm, out_hbm, idx_v, vals_v,
                  rows_a, rows_b, out_a, out_b, sem_a, sem_b, sem_oa, sem_ob):
        wid = lax.axis_index("s") * SC_NC + lax.axis_index("c")
        base = wid * per_w
        pltpu.sync_copy(idx_hbm.at[pl.ds(base * K, per_w * K)], idx_v)
        pltpu.sync_copy(vals_hbm.at[pl.ds(base * K, per_w * K)], vals_v)

        def gsrc(g):
            return hs_hbm.at[idx_v.at[pl.ds(g * (G * K), G * K)]]

        def start(g, buf, sem):
            pltpu.async_copy(gsrc(g), buf, sem)

        def wait(g, buf, sem):
            pltpu.make_async_copy(gsrc(g), buf, sem).wait()

        def odst(g):
            return out_hbm.at[pl.ds(base + g * G, G)]

        def compute(g, buf, obuf):
            def crow(r, carry):
                wv = vals_v[pl.ds((g * G + r) * K, K)]
                accs = [jnp.zeros((16,), jnp.float32) for _ in range(nchunk)]
                for j in range(K):
                    wjv = jnp.broadcast_to(wv[j], (16,))
                    for c in range(nchunk):
                        accs[c] = accs[c] + wjv * buf[r * K + j,
                                                      pl.ds(c * 16, 16)]
                for c in range(nchunk):
                    obuf[r, pl.ds(c * 16, 16)] = accs[c]
                return carry

            lax.fori_loop(0, G, crow, 0)

        start(0, rows_a, sem_a)

        def body(i, carry):
            g0 = 2 * i
            g1 = g0 + 1
            start(g1, rows_b, sem_b)
            wait(g0, rows_a, sem_a)

            @pl.when(i > 0)
            def _():
                pltpu.make_async_copy(out_a, odst(0), sem_oa).wait()

            compute(g0, rows_a, out_a)
            pltpu.async_copy(out_a, odst(g0), sem_oa)

            @pl.when(i < npairs - 1)
            def _():
                start(g0 + 2, rows_a, sem_a)

            wait(g1, rows_b, sem_b)

            @pl.when(i > 0)
            def _():
                pltpu.make_async_copy(out_b, odst(0), sem_ob).wait()

            compute(g1, rows_b, out_b)
            pltpu.async_copy(out_b, odst(g1), sem_ob)
            return carry

        lax.fori_loop(0, npairs, body, 0)
        pltpu.make_async_copy(out_a, odst(0), sem_oa).wait()
        pltpu.make_async_copy(out_b, odst(0), sem_ob).wait()

    return sc_kernel(hs, idxf, valsf)


def _partial_stats(z):
    rb = z.shape[0]
    s = z[0:8, :]
    q = z[0:8, :] * z[0:8, :]
    for i in range(1, rb // 8):
        blk = z[i * 8:(i + 1) * 8, :]
        s = s + blk
        q = q + blk * blk
    return s, q


def _bn_apply(z, s8_ref, q8_ref, g_ref, b_ref, n):
    mu = jnp.sum(s8_ref[...], axis=0, keepdims=True) / n
    ex2 = jnp.sum(q8_ref[...], axis=0, keepdims=True) / n
    var = ex2 - mu * mu
    inv = lax.rsqrt(var + 1e-5)
    return (z - mu) * inv * g_ref[...] + b_ref[...]


def _leaky(h):
    return jnp.where(h >= 0, h, 0.01 * h)


def _c1_body(nrows, g_ref, hs_ref, dinv_ref, b_ref, z_ref, s_ref, q_ref):
    z = dinv_ref[...] * (g_ref[...] + hs_ref[...]) + b_ref[...]
    z_ref[...] = z
    s, q = _partial_stats(z)

    @pl.when(pl.program_id(0) == 0)
    def _():
        s_ref[...] = jnp.zeros_like(s_ref)
        q_ref[...] = jnp.zeros_like(q_ref)

    s_ref[...] += s
    q_ref[...] += q


def _c1(g, hs, dinv, b2d):
    n, hid = g.shape
    rb = _row_block(n)
    grid = n // rb
    return pl.pallas_call(
        functools.partial(_c1_body, n),
        grid=(grid,),
        in_specs=[
            pl.BlockSpec((rb, hid), lambda i: (i, 0)),
            pl.BlockSpec((rb, hid), lambda i: (i, 0)),
            pl.BlockSpec((rb, 1), lambda i: (i, 0)),
            pl.BlockSpec((1, hid), lambda i: (0, 0)),
        ],
        out_specs=[
            pl.BlockSpec((rb, hid), lambda i: (i, 0)),
            pl.BlockSpec((8, hid), lambda i: (0, 0)),
            pl.BlockSpec((8, hid), lambda i: (0, 0)),
        ],
        out_shape=[
            jax.ShapeDtypeStruct((n, hid), jnp.float32),
            jax.ShapeDtypeStruct((8, hid), jnp.float32),
            jax.ShapeDtypeStruct((8, hid), jnp.float32),
        ],
    )(g, hs, dinv, b2d)


def _cmid_body(nrows, z_ref, s8_ref, q8_ref, g_ref, b_ref, w_ref, wb_ref,
               z2_ref, s_ref, q_ref):
    a = _leaky(_bn_apply(z_ref[...], s8_ref, q8_ref, g_ref, b_ref, nrows))
    z2 = jnp.dot(a, w_ref[...], preferred_element_type=jnp.float32) + wb_ref[...]
    z2_ref[...] = z2
    s, q = _partial_stats(z2)

    @pl.when(pl.program_id(0) == 0)
    def _():
        s_ref[...] = jnp.zeros_like(s_ref)
        q_ref[...] = jnp.zeros_like(q_ref)

    s_ref[...] += s
    q_ref[...] += q


def _cmid(z, s8, q8, bn_g, bn_b, w, wb):
    n, cin = z.shape
    cout = w.shape[1]
    rb = _row_block(n)
    grid = n // rb
    return pl.pallas_call(
        functools.partial(_cmid_body, n),
        grid=(grid,),
        in_specs=[
            pl.BlockSpec((rb, cin), lambda i: (i, 0)),
            pl.BlockSpec((8, cin), lambda i: (0, 0)),
            pl.BlockSpec((8, cin), lambda i: (0, 0)),
            pl.BlockSpec((1, cin), lambda i: (0, 0)),
            pl.BlockSpec((1, cin), lambda i: (0, 0)),
            pl.BlockSpec((cin, cout), lambda i: (0, 0)),
            pl.BlockSpec((1, cout), lambda i: (0, 0)),
        ],
        out_specs=[
            pl.BlockSpec((rb, cout), lambda i: (i, 0)),
            pl.BlockSpec((8, cout), lambda i: (0, 0)),
            pl.BlockSpec((8, cout), lambda i: (0, 0)),
        ],
        out_shape=[
            jax.ShapeDtypeStruct((n, cout), jnp.float32),
            jax.ShapeDtypeStruct((8, cout), jnp.float32),
            jax.ShapeDtypeStruct((8, cout), jnp.float32),
        ],
    )(z, s8, q8, bn_g, bn_b, w, wb)


def _c5_body(nrows, nblocks, z_ref, s8_ref, q8_ref, g_ref, b_ref,
             w4_ref, b4_ref, tbw_ref, tbb_ref, tfw_ref, tfb_ref,
             bkw_ref, bkb_ref, fow_ref, fob_ref, back_ref, fore_ref):
    residual = _leaky(_bn_apply(z_ref[...], s8_ref, q8_ref, g_ref, b_ref,
                                nrows))
    rb = residual.shape[0]
    out = fow_ref.shape[2]
    forecast = jnp.zeros((rb, out), jnp.float32)
    for bi in range(nblocks):
        h = residual
        for li in range(4):
            k = bi * 4 + li
            h = jnp.dot(h, w4_ref[k], preferred_element_type=jnp.float32)
            h = jnp.maximum(h + b4_ref[k], 0.0)
        tb = jnp.dot(h, tbw_ref[bi], preferred_element_type=jnp.float32) + tbb_ref[bi]
        tf = jnp.dot(h, tfw_ref[bi], preferred_element_type=jnp.float32) + tfb_ref[bi]
        bc = jnp.dot(tb, bkw_ref[bi], preferred_element_type=jnp.float32) + bkb_ref[bi]
        fc = jnp.dot(tf, fow_ref[bi], preferred_element_type=jnp.float32) + fob_ref[bi]
        residual = residual - bc
        forecast = forecast + fc
    back_ref[...] = residual
    fore_ref[...] = forecast


def _c5(z, s8, q8, bn_g, bn_b, w4, b4, tbw, tbb, tfw, tfb, bkw, bkb, fow, fob):
    n, seq = z.shape
    nblocks = tbw.shape[0]
    theta = tbw.shape[2]
    out = fow.shape[2]
    rb = _row_block(n)
    grid = n // rb
    full = lambda a: pl.BlockSpec(a.shape, lambda i: (0,) * a.ndim)
    return pl.pallas_call(
        functools.partial(_c5_body, n, nblocks),
        grid=(grid,),
        in_specs=[
            pl.BlockSpec((rb, seq), lambda i: (i, 0)),
            pl.BlockSpec((8, seq), lambda i: (0, 0)),
            pl.BlockSpec((8, seq), lambda i: (0, 0)),
            pl.BlockSpec((1, seq), lambda i: (0, 0)),
            pl.BlockSpec((1, seq), lambda i: (0, 0)),
            full(w4), full(b4), full(tbw), full(tbb), full(tfw), full(tfb),
            full(bkw), full(bkb), full(fow), full(fob),
        ],
        out_specs=[
            pl.BlockSpec((rb, seq), lambda i: (i, 0)),
            pl.BlockSpec((rb, out), lambda i: (i, 0)),
        ],
        out_shape=[
            jax.ShapeDtypeStruct((n, seq), jnp.float32),
            jax.ShapeDtypeStruct((n, out), jnp.float32),
        ],
    )(z, s8, q8, bn_g, bn_b, w4, b4, tbw, tbb, tfw, tfb, bkw, bkb, fow, fob)


def kernel(x, params):
    p = params
    n = x.shape[0]
    r2 = lambda v: v.reshape(1, -1)

    xn = x / (jnp.linalg.norm(x, axis=1, keepdims=True) + 1e-12)
    vals, idx, dinv, hs = _topk(xn, x, p['gcn_w'])

    np_rows = ((n + SC_NW * 8 - 1) // (SC_NW * 8)) * (SC_NW * 8)
    pad = np_rows - n
    idxf = jnp.concatenate(
        [idx, jnp.zeros((pad, K), jnp.int32)], axis=0).reshape(-1)
    valsf = jnp.concatenate(
        [vals, jnp.zeros((pad, K), jnp.float32)], axis=0).reshape(-1)
    g = _gcn_gather(hs, idxf, valsf, np_rows)[:n]

    z0, s0, q0 = _c1(g, hs, dinv, r2(p['gcn_b']))
    z1, s1, q1 = _cmid(z0, s0, q0, r2(p['bn0_g']), r2(p['bn0_b']),
                       p['fc1_w'], r2(p['fc1_b']))
    z2, s2, q2 = _cmid(z1, s1, q1, r2(p['bn1_g']), r2(p['bn1_b']),
                       p['fc2_w'], r2(p['fc2_b']))
    z3, s3, q3 = _cmid(z2, s2, q2, r2(p['bn2_g']), r2(p['bn2_b']),
                       p['fc3_w'], r2(p['fc3_b']))

    blocks = p['blocks']
    nb = len(blocks)
    w4 = jnp.stack([blk['w%d' % li] for blk in blocks for li in range(4)])
    b4 = jnp.stack([blk['b%d' % li] for blk in blocks
                    for li in range(4)])[:, None, :]
    tbw = jnp.stack([blk['tb_w'] for blk in blocks])
    tbb = jnp.stack([blk['tb_b'] for blk in blocks])[:, None, :]
    tfw = jnp.stack([blk['tf_w'] for blk in blocks])
    tfb = jnp.stack([blk['tf_b'] for blk in blocks])[:, None, :]
    bkw = jnp.stack([blk['back_w'] for blk in blocks])
    bkb = jnp.stack([blk['back_b'] for blk in blocks])[:, None, :]
    fow = jnp.stack([blk['fore_w'] for blk in blocks])
    fob = jnp.stack([blk['fore_b'] for blk in blocks])[:, None, :]

    backcast, forecast = _c5(z3, s3, q3, r2(p['bn3_g']), r2(p['bn3_b']),
                             w4, b4, tbw, tbb, tfw, tfb, bkw, bkb, fow, fob)
    return backcast, forecast


# speedup vs baseline: 1.0970x; 1.0970x over previous
import functools

import jax
import jax.numpy as jnp
from jax import lax
from jax.experimental import pallas as pl
from jax.experimental.pallas import tpu as pltpu
from jax.experimental.pallas import tpu_sc as plsc

K = 16
NEG = -3.0e38
IBIG = 2147483647

SC_NC = 2
SC_NS = 16
SC_NW = SC_NC * SC_NS


def _row_block(n):
    for rb in (400, 200, 80, 40, 16, 8):
        if n % rb == 0:
            return rb
    return n


def _topk_body(xn_ref, xnall_ref, x_ref, w_ref, vals_ref, idx_ref,
               dinv_ref, hs_ref):
    xn = xn_ref[...]
    sim = lax.dot_general(
        xn, xnall_ref[...],
        dimension_numbers=(((1,), (1,)), ((), ())),
        preferred_element_type=jnp.float32,
    )
    b, n = sim.shape
    colid = lax.broadcasted_iota(jnp.int32, (b, n), 1)
    vcols = []
    icols = []
    for _ in range(K):
        m = jnp.max(sim, axis=1, keepdims=True)
        cand = jnp.where(sim == m, colid, IBIG)
        ix = jnp.min(cand, axis=1, keepdims=True)
        sim = jnp.where(colid == ix, NEG, sim)
        vcols.append(m)
        icols.append(ix)
    vals = jnp.concatenate(vcols, axis=1)
    idx = jnp.concatenate(icols, axis=1)
    vals_ref[...] = vals
    idx_ref[...] = idx
    deg = 1.0 + jnp.sum(vals, axis=1, keepdims=True)
    dinv = jnp.where(deg > 0, lax.rsqrt(deg), 0.0)
    dinv_ref[...] = dinv
    h = jnp.dot(x_ref[...], w_ref[...], preferred_element_type=jnp.float32)
    hs_ref[...] = dinv * h


def _topk(xn, x, w):
    n, seq = xn.shape
    hid = w.shape[1]
    rb = 200 if n % 200 == 0 else _row_block(n)
    grid = n // rb
    return pl.pallas_call(
        _topk_body,
        grid=(grid,),
        in_specs=[
            pl.BlockSpec((rb, seq), lambda i: (i, 0)),
            pl.BlockSpec((n, seq), lambda i: (0, 0)),
            pl.BlockSpec((rb, seq), lambda i: (i, 0)),
            pl.BlockSpec((seq, hid), lambda i: (0, 0)),
        ],
        out_specs=[
            pl.BlockSpec((rb, K), lambda i: (i, 0)),
            pl.BlockSpec((rb, K), lambda i: (i, 0)),
            pl.BlockSpec((rb, 1), lambda i: (i, 0)),
            pl.BlockSpec((rb, hid), lambda i: (i, 0)),
        ],
        out_shape=[
            jax.ShapeDtypeStruct((n, K), jnp.float32),
            jax.ShapeDtypeStruct((n, K), jnp.int32),
            jax.ShapeDtypeStruct((n, 1), jnp.float32),
            jax.ShapeDtypeStruct((n, hid), jnp.float32),
        ],
    )(xn, xn, x, w)


def _gcn_gather(hs, idxf, valsf, np_rows):
    d = hs.shape[1]
    per_w = np_rows // SC_NW
    G = 4
    ngroups = per_w // G
    npairs = ngroups // 2
    nchunk = d // 16
    mesh = plsc.VectorSubcoreMesh(core_axis_name="c", subcore_axis_name="s")

    @functools.partial(
        pl.kernel,
        mesh=mesh,
        out_type=jax.ShapeDtypeStruct((np_rows, d), jnp.float32),
        scratch_types=[
            pltpu.VMEM((per_w * K,), jnp.int32),
            pltpu.VMEM((per_w * K,), jnp.float32),
            pltpu.VMEM((G * K, d), jnp.float32),
            pltpu.VMEM((G * K, d), jnp.float32),
            pltpu.VMEM((G, d), jnp.float32),
            pltpu.VMEM((G, d), jnp.float32),
            pltpu.SemaphoreType.DMA,
            pltpu.SemaphoreType.DMA,
            pltpu.SemaphoreType.DMA,
            pltpu.SemaphoreType.DMA,
        ],
    )
    def sc_kernel(hs_hbm, idx_hbm, vals_hbm, out_hbm, idx_v, vals_v,
                  rows_a, rows_b, out_a, out_b, sem_a, sem_b, sem_oa, sem_ob):
        wid = lax.axis_index("s") * SC_NC + lax.axis_index("c")
        base = wid * per_w
        pltpu.sync_copy(idx_hbm.at[pl.ds(base * K, per_w * K)], idx_v)
        pltpu.sync_copy(vals_hbm.at[pl.ds(base * K, per_w * K)], vals_v)

        def gsrc(g):
            return hs_hbm.at[idx_v.at[pl.ds(g * (G * K), G * K)]]

        def start(g, buf, sem):
            pltpu.async_copy(gsrc(g), buf, sem)

        def wait(g, buf, sem):
            pltpu.make_async_copy(gsrc(g), buf, sem).wait()

        def odst(g):
            return out_hbm.at[pl.ds(base + g * G, G)]

        def compute(g, buf, obuf):
            def crow(r, carry):
                wv = vals_v[pl.ds((g * G + r) * K, K)]
                accs = [jnp.zeros((16,), jnp.float32) for _ in range(nchunk)]
                for j in range(K):
                    wjv = jnp.broadcast_to(wv[j], (16,))
                    for c in range(nchunk):
                        accs[c] = accs[c] + wjv * buf[r * K + j,
                                                      pl.ds(c * 16, 16)]
                for c in range(nchunk):
                    obuf[r, pl.ds(c * 16, 16)] = accs[c]
                return carry

            lax.fori_loop(0, G, crow, 0)

        start(0, rows_a, sem_a)

        def body(i, carry):
            g0 = 2 * i
            g1 = g0 + 1
            start(g1, rows_b, sem_b)
            wait(g0, rows_a, sem_a)

            @pl.when(i > 0)
            def _():
                pltpu.make_async_copy(out_a, odst(0), sem_oa).wait()

            compute(g0, rows_a, out_a)
            pltpu.async_copy(out_a, odst(g0), sem_oa)

            @pl.when(i < npairs - 1)
            def _():
                start(g0 + 2, rows_a, sem_a)

            wait(g1, rows_b, sem_b)

            @pl.when(i > 0)
            def _():
                pltpu.make_async_copy(out_b, odst(0), sem_ob).wait()

            compute(g1, rows_b, out_b)
            pltpu.async_copy(out_b, odst(g1), sem_ob)
            return carry

        lax.fori_loop(0, npairs, body, 0)
        pltpu.make_async_copy(out_a, odst(0), sem_oa).wait()
        pltpu.make_async_copy(out_b, odst(0), sem_ob).wait()

    return sc_kernel(hs, idxf, valsf)


def _partial_stats(z):
    rb = z.shape[0]
    s = z[0:8, :]
    q = z[0:8, :] * z[0:8, :]
    for i in range(1, rb // 8):
        blk = z[i * 8:(i + 1) * 8, :]
        s = s + blk
        q = q + blk * blk
    return s, q


def _bn_apply(z, s8_ref, q8_ref, g_ref, b_ref, n):
    mu = jnp.sum(s8_ref[...], axis=0, keepdims=True) / n
    ex2 = jnp.sum(q8_ref[...], axis=0, keepdims=True) / n
    var = ex2 - mu * mu
    inv = lax.rsqrt(var + 1e-5)
    return (z - mu) * inv * g_ref[...] + b_ref[...]


def _leaky(h):
    return jnp.where(h >= 0, h, 0.01 * h)


def _c1_body(nrows, g_ref, hs_ref, dinv_ref, b_ref, z_ref, s_ref, q_ref):
    z = dinv_ref[...] * (g_ref[...] + hs_ref[...]) + b_ref[...]
    z_ref[...] = z
    s, q = _partial_stats(z)

    @pl.when(pl.program_id(0) == 0)
    def _():
        s_ref[...] = jnp.zeros_like(s_ref)
        q_ref[...] = jnp.zeros_like(q_ref)

    s_ref[...] += s
    q_ref[...] += q


def _c1(g, hs, dinv, b2d):
    n, hid = g.shape
    rb = _row_block(n)
    grid = n // rb
    return pl.pallas_call(
        functools.partial(_c1_body, n),
        grid=(grid,),
        in_specs=[
            pl.BlockSpec((rb, hid), lambda i: (i, 0)),
            pl.BlockSpec((rb, hid), lambda i: (i, 0)),
            pl.BlockSpec((rb, 1), lambda i: (i, 0)),
            pl.BlockSpec((1, hid), lambda i: (0, 0)),
        ],
        out_specs=[
            pl.BlockSpec((rb, hid), lambda i: (i, 0)),
            pl.BlockSpec((8, hid), lambda i: (0, 0)),
            pl.BlockSpec((8, hid), lambda i: (0, 0)),
        ],
        out_shape=[
            jax.ShapeDtypeStruct((n, hid), jnp.float32),
            jax.ShapeDtypeStruct((8, hid), jnp.float32),
            jax.ShapeDtypeStruct((8, hid), jnp.float32),
        ],
    )(g, hs, dinv, b2d)


def _cmid_body(nrows, z_ref, s8_ref, q8_ref, g_ref, b_ref, w_ref, wb_ref,
               z2_ref, s_ref, q_ref):
    a = _leaky(_bn_apply(z_ref[...], s8_ref, q8_ref, g_ref, b_ref, nrows))
    z2 = jnp.dot(a, w_ref[...], preferred_element_type=jnp.float32) + wb_ref[...]
    z2_ref[...] = z2
    s, q = _partial_stats(z2)

    @pl.when(pl.program_id(0) == 0)
    def _():
        s_ref[...] = jnp.zeros_like(s_ref)
        q_ref[...] = jnp.zeros_like(q_ref)

    s_ref[...] += s
    q_ref[...] += q


def _cmid(z, s8, q8, bn_g, bn_b, w, wb):
    n, cin = z.shape
    cout = w.shape[1]
    rb = _row_block(n)
    grid = n // rb
    return pl.pallas_call(
        functools.partial(_cmid_body, n),
        grid=(grid,),
        in_specs=[
            pl.BlockSpec((rb, cin), lambda i: (i, 0)),
            pl.BlockSpec((8, cin), lambda i: (0, 0)),
            pl.BlockSpec((8, cin), lambda i: (0, 0)),
            pl.BlockSpec((1, cin), lambda i: (0, 0)),
            pl.BlockSpec((1, cin), lambda i: (0, 0)),
            pl.BlockSpec((cin, cout), lambda i: (0, 0)),
            pl.BlockSpec((1, cout), lambda i: (0, 0)),
        ],
        out_specs=[
            pl.BlockSpec((rb, cout), lambda i: (i, 0)),
            pl.BlockSpec((8, cout), lambda i: (0, 0)),
            pl.BlockSpec((8, cout), lambda i: (0, 0)),
        ],
        out_shape=[
            jax.ShapeDtypeStruct((n, cout), jnp.float32),
            jax.ShapeDtypeStruct((8, cout), jnp.float32),
            jax.ShapeDtypeStruct((8, cout), jnp.float32),
        ],
    )(z, s8, q8, bn_g, bn_b, w, wb)


def _c5_body(nrows, nblocks, z_ref, s8_ref, q8_ref, g_ref, b_ref,
             w4_ref, b4_ref, tbw_ref, tbb_ref, tfw_ref, tfb_ref,
             bkw_ref, bkb_ref, fow_ref, fob_ref, back_ref, fore_ref):
    residual = _leaky(_bn_apply(z_ref[...], s8_ref, q8_ref, g_ref, b_ref,
                                nrows))
    rb = residual.shape[0]
    out = fow_ref.shape[2]
    forecast = jnp.zeros((rb, out), jnp.float32)
    for bi in range(nblocks):
        h = residual
        for li in range(4):
            k = bi * 4 + li
            h = jnp.dot(h, w4_ref[k], preferred_element_type=jnp.float32)
            h = jnp.maximum(h + b4_ref[k], 0.0)
        tb = jnp.dot(h, tbw_ref[bi], preferred_element_type=jnp.float32) + tbb_ref[bi]
        tf = jnp.dot(h, tfw_ref[bi], preferred_element_type=jnp.float32) + tfb_ref[bi]
        bc = jnp.dot(tb, bkw_ref[bi], preferred_element_type=jnp.float32) + bkb_ref[bi]
        fc = jnp.dot(tf, fow_ref[bi], preferred_element_type=jnp.float32) + fob_ref[bi]
        residual = residual - bc
        forecast = forecast + fc
    back_ref[...] = residual
    fore_ref[...] = forecast


def _c5(z, s8, q8, bn_g, bn_b, w4, b4, tbw, tbb, tfw, tfb, bkw, bkb, fow, fob):
    n, seq = z.shape
    nblocks = tbw.shape[0]
    theta = tbw.shape[2]
    out = fow.shape[2]
    rb = _row_block(n)
    grid = n // rb
    full = lambda a: pl.BlockSpec(a.shape, lambda i: (0,) * a.ndim)
    return pl.pallas_call(
        functools.partial(_c5_body, n, nblocks),
        grid=(grid,),
        in_specs=[
            pl.BlockSpec((rb, seq), lambda i: (i, 0)),
            pl.BlockSpec((8, seq), lambda i: (0, 0)),
            pl.BlockSpec((8, seq), lambda i: (0, 0)),
            pl.BlockSpec((1, seq), lambda i: (0, 0)),
            pl.BlockSpec((1, seq), lambda i: (0, 0)),
            full(w4), full(b4), full(tbw), full(tbb), full(tfw), full(tfb),
            full(bkw), full(bkb), full(fow), full(fob),
        ],
        out_specs=[
            pl.BlockSpec((rb, seq), lambda i: (i, 0)),
            pl.BlockSpec((rb, out), lambda i: (i, 0)),
        ],
        out_shape=[
            jax.ShapeDtypeStruct((n, seq), jnp.float32),
            jax.ShapeDtypeStruct((n, out), jnp.float32),
        ],
    )(z, s8, q8, bn_g, bn_b, w4, b4, tbw, tbb, tfw, tfb, bkw, bkb, fow, fob)


def kernel(x, params):
    p = params
    n = x.shape[0]
    r2 = lambda v: v.reshape(1, -1)

    xn = x / (jnp.linalg.norm(x, axis=1, keepdims=True) + 1e-12)
    vals, idx, dinv, hs = _topk(xn, x, p['gcn_w'])

    np_rows = ((n + SC_NW * 8 - 1) // (SC_NW * 8)) * (SC_NW * 8)
    pad = np_rows - n
    idxf = jnp.concatenate(
        [idx, jnp.zeros((pad, K), jnp.int32)], axis=0).reshape(-1)
    valsf = jnp.concatenate(
        [vals, jnp.zeros((pad, K), jnp.float32)], axis=0).reshape(-1)
    g = _gcn_gather(hs, idxf, valsf, np_rows)[:n]

    z0, s0, q0 = _c1(g, hs, dinv, r2(p['gcn_b']))
    z1, s1, q1 = _cmid(z0, s0, q0, r2(p['bn0_g']), r2(p['bn0_b']),
                       p['fc1_w'], r2(p['fc1_b']))
    z2, s2, q2 = _cmid(z1, s1, q1, r2(p['bn1_g']), r2(p['bn1_b']),
                       p['fc2_w'], r2(p['fc2_b']))
    z3, s3, q3 = _cmid(z2, s2, q2, r2(p['bn2_g']), r2(p['bn2_b']),
                       p['fc3_w'], r2(p['fc3_b']))

    blocks = p['blocks']
    nb = len(blocks)
    w4 = jnp.stack([blk['w%d' % li] for blk in blocks for li in range(4)])
    b4 = jnp.stack([blk['b%d' % li] for blk in blocks
                    for li in range(4)])[:, None, :]
    tbw = jnp.stack([blk['tb_w'] for blk in blocks])
    tbb = jnp.stack([blk['tb_b'] for blk in blocks])[:, None, :]
    tfw = jnp.stack([blk['tf_w'] for blk in blocks])
    tfb = jnp.stack([blk['tf_b'] for blk in blocks])[:, None, :]
    bkw = jnp.stack([blk['back_w'] for blk in blocks])
    bkb = jnp.stack([blk['back_b'] for blk in blocks])[:, None, :]
    fow = jnp.stack([blk['fore_w'] for blk in blocks])
    fob = jnp.stack([blk['fore_b'] for blk in blocks])[:, None, :]

    backcast, forecast = _c5(z3, s3, q3, r2(p['bn3_g']), r2(p['bn3_b']),
                             w4, b4, tbw, tbb, tfw, tfb, bkw, bkb, fow, fob)
    return backcast, forecast
